# baseline (speedup 1.0000x reference)
import functools

import jax
import jax.numpy as jnp
from jax import lax
from jax.experimental import pallas as pl
from jax.experimental.pallas import tpu as pltpu
from jax.experimental.pallas import tpu_sc as plsc

CTX_DIM = 768
N_CLS = 1000
SEQ = 77
N_CTX = 4
LANES = 16
CTX_FLAT = N_CTX * CTX_DIM


def _sc_prompts(idx3, token_embedding, ctx_flat):
    info = plsc.get_sparse_core_info()
    nw = info.num_cores * info.num_subcores
    base = N_CLS // nw
    extra = N_CLS - base * nw
    mesh = plsc.VectorSubcoreMesh(core_axis_name="c", subcore_axis_name="s")

    @functools.partial(
        pl.kernel,
        mesh=mesh,
        out_type=jax.ShapeDtypeStruct((N_CLS, SEQ, CTX_DIM), jnp.float32),
        scratch_types=[
            pltpu.VMEM((2, 1, SEQ), jnp.int32),
            pltpu.VMEM((CTX_FLAT,), jnp.float32),
            pltpu.VMEM((2, SEQ, CTX_DIM), jnp.float32),
            pltpu.SemaphoreType.DMA,
            pltpu.SemaphoreType.DMA,
            pltpu.SemaphoreType.DMA,
        ],
    )
    def k(idx_hbm, table_hbm, ctx_hbm, out_hbm,
          idx_v, ctx_v, buf_v, sem_i, sem_g, sem_o):
        wid = lax.axis_index("s") * info.num_cores + lax.axis_index("c")
        start = wid * base + jnp.minimum(wid, extra)
        count = base + jnp.where(wid < extra, 1, 0)

        pltpu.sync_copy(ctx_hbm, ctx_v)
        pltpu.sync_copy(idx_hbm.at[start], idx_v.at[0])

        def body(i, carry):
            b = lax.rem(i, 2)
            c = start + i

            @pl.when(i >= 1)
            def _():
                pltpu.make_async_copy(
                    idx_hbm.at[c], idx_v.at[b], sem_i).wait()

            @pl.when(i >= 2)
            def _():
                pltpu.make_async_copy(
                    buf_v.at[b], out_hbm.at[c], sem_o).wait()

            gh = pltpu.async_copy(
                table_hbm.at[idx_v.at[b, 0]], buf_v.at[b], sem_g)

            @pl.when(i + 1 < count)
            def _():
                pltpu.async_copy(idx_hbm.at[c + 1], idx_v.at[1 - b], sem_i)

            gh.wait()
            for r in range(N_CTX):
                for j in range(CTX_DIM // LANES):
                    buf_v[b, 1 + r, pl.ds(j * LANES, LANES)] = (
                        ctx_v[pl.ds(r * CTX_DIM + j * LANES, LANES)])
            pltpu.async_copy(buf_v.at[b], out_hbm.at[c], sem_o)
            return carry

        lax.fori_loop(0, count, body, 0)

        @pl.when(count >= 1)
        def _():
            pltpu.make_async_copy(
                buf_v.at[0], out_hbm.at[start], sem_o).wait()

        @pl.when(count >= 2)
        def _():
            pltpu.make_async_copy(
                buf_v.at[1], out_hbm.at[start], sem_o).wait()

    return k(idx3, token_embedding, ctx_flat)


def kernel(tokenized_prompts, token_embedding, ctx):
    z = jnp.zeros((N_CLS, N_CTX), jnp.int32)
    idx3 = jnp.concatenate(
        [tokenized_prompts[:, :1], z, tokenized_prompts[:, 1 + N_CTX:]],
        axis=1).reshape(N_CLS, 1, SEQ)
    return _sc_prompts(idx3, token_embedding, ctx.reshape(CTX_FLAT))


# speedup vs baseline: 1.8086x; 1.8086x over previous
import functools

import jax
import jax.numpy as jnp
from jax import lax
from jax.experimental import pallas as pl
from jax.experimental.pallas import tpu as pltpu
from jax.experimental.pallas import tpu_sc as plsc

CTX_DIM = 768
N_CLS = 1000
SEQ = 77
N_CTX = 4
LANES = 16
CTX_FLAT = N_CTX * CTX_DIM


def _sc_prompts(idx3, token_embedding, ctx_flat):
    info = plsc.get_sparse_core_info()
    nw = info.num_cores * info.num_subcores
    base = N_CLS // nw
    extra = N_CLS - base * nw
    mesh = plsc.VectorSubcoreMesh(core_axis_name="c", subcore_axis_name="s")

    @functools.partial(
        pl.kernel,
        mesh=mesh,
        out_type=jax.ShapeDtypeStruct((N_CLS, SEQ, CTX_DIM), jnp.float32),
        scratch_types=[
            pltpu.VMEM((2, 1, SEQ), jnp.int32),
            pltpu.VMEM((CTX_FLAT,), jnp.float32),
            pltpu.VMEM((2, SEQ, CTX_DIM), jnp.float32),
            pltpu.SemaphoreType.DMA,
            pltpu.SemaphoreType.DMA,
            pltpu.SemaphoreType.DMA,
        ],
    )
    def k(idx_hbm, table_hbm, ctx_hbm, out_hbm,
          idx_v, ctx_v, buf_v, sem_i, sem_g, sem_o):
        wid = lax.axis_index("s") * info.num_cores + lax.axis_index("c")
        start = wid * base + jnp.minimum(wid, extra)
        count = base + jnp.where(wid < extra, 1, 0)

        pltpu.sync_copy(ctx_hbm, ctx_v)
        pltpu.sync_copy(idx_hbm.at[start], idx_v.at[0])

        def body(i, carry):
            b = lax.rem(i, 2)
            c = start + i

            @pl.when(i >= 1)
            def _():
                pltpu.make_async_copy(
                    idx_hbm.at[c], idx_v.at[b], sem_i).wait()

            @pl.when(i >= 2)
            def _():
                pltpu.make_async_copy(
                    buf_v.at[b], out_hbm.at[c], sem_o).wait()

            gh = pltpu.async_copy(
                table_hbm.at[idx_v.at[b, 0]], buf_v.at[b], sem_g)

            @pl.when(i + 1 < count)
            def _():
                pltpu.async_copy(idx_hbm.at[c + 1], idx_v.at[1 - b], sem_i)

            gh.wait()
            for r in range(N_CTX):
                for j in range(CTX_DIM // LANES):
                    buf_v[b, 1 + r, pl.ds(j * LANES, LANES)] = (
                        ctx_v[pl.ds(r * CTX_DIM + j * LANES, LANES)])
            pltpu.async_copy(buf_v.at[b], out_hbm.at[c], sem_o)
            return carry

        lax.fori_loop(0, count, body, 0)

        @pl.when(count >= 1)
        def _():
            pltpu.make_async_copy(
                buf_v.at[0], out_hbm.at[start], sem_o).wait()

        @pl.when(count >= 2)
        def _():
            pltpu.make_async_copy(
                buf_v.at[1], out_hbm.at[start], sem_o).wait()

    return k(idx3, token_embedding, ctx_flat)


def kernel(tokenized_prompts, token_embedding, ctx):
    idx3 = tokenized_prompts.reshape(N_CLS, 1, SEQ)
    return _sc_prompts(idx3, token_embedding, ctx.reshape(CTX_FLAT))
